# baseline (speedup 1.0000x reference)
import functools

import jax
import jax.numpy as jnp
from jax import lax
from jax.experimental import pallas as pl
from jax.experimental.pallas import tpu as pltpu
from jax.experimental.pallas import tpu_sc as plsc

N = 10000
D = 256
E = 160000
NP = 10240
EP = 163840
CH = 128
NT = 16
ROWS_PT = NP // NT


def _deg_body(dstr, out_hbm, acc, dst_a, tmp, ones_v, sem):
    del sem
    c = lax.axis_index("c")
    t = lax.axis_index("s")
    w = c * NT + t

    def _fill_zero(i, _):
        tmp[pl.ds(i * 16, 16)] = jnp.zeros((16,), jnp.float32)
        return 0

    lax.fori_loop(0, ROWS_PT // 16, _fill_zero, 0)

    def _fill_one(i, _):
        ones_v[pl.ds(i * 16, 16)] = jnp.ones((16,), jnp.float32)
        return 0

    lax.fori_loop(0, CH // 16, _fill_one, 0)

    pltpu.sync_copy(tmp, acc.at[pl.ds(t * ROWS_PT, ROWS_PT)])
    pltpu.sync_copy(dstr.at[pl.ds(w * 40, 40)], dst_a)
    plsc.subcore_barrier()

    def _scat(j, _):
        pltpu.sync_copy(ones_v, acc.at[dst_a.at[j]], add=True)
        return 0

    lax.fori_loop(0, 40, _scat, 0)
    plsc.subcore_barrier()

    pltpu.sync_copy(acc.at[pl.ds(t * ROWS_PT, ROWS_PT)], tmp)
    pltpu.sync_copy(tmp, out_hbm.at[pl.ds(c * NP + t * ROWS_PT, ROWS_PT)])


def _agg_body(y_hbm, soff_hbm, dstf, zeros_hbm, out_hbm, acc, src_b, dst_b,
              rows, sem_g, sem_s, sem_si, sem_di):
    c = lax.axis_index("c")
    t = lax.axis_index("s")
    NCH = 80

    def _issue_idx(s, k):
        r = lax.rem(k, 4)
        sbase = (s * 2 + c) * EP
        eoff = (t * NCH + k) * CH
        pltpu.async_copy(soff_hbm.at[pl.ds(sbase + eoff, CH)],
                         src_b.at[r], sem_si.at[r])
        pltpu.async_copy(dstf.at[pl.ds(eoff, CH)], dst_b.at[r], sem_di.at[r])

    def _wait_idx(k):
        r = lax.rem(k, 4)
        pltpu.make_async_copy(soff_hbm.at[pl.ds(0, CH)], src_b.at[r],
                              sem_si.at[r]).wait()
        pltpu.make_async_copy(dstf.at[pl.ds(0, CH)], dst_b.at[r],
                              sem_di.at[r]).wait()

    def _issue_gather(k):
        r = lax.rem(k, 4)
        rb = lax.rem(k, 2)
        pltpu.async_copy(y_hbm.at[src_b.at[r]], rows.at[rb], sem_g.at[rb])

    def _wait_gather(k):
        r = lax.rem(k, 4)
        rb = lax.rem(k, 2)
        pltpu.make_async_copy(y_hbm.at[src_b.at[r]], rows.at[rb],
                              sem_g.at[rb]).wait()

    def _issue_scatter(k):
        r = lax.rem(k, 4)
        rb = lax.rem(k, 2)
        pltpu.async_copy(rows.at[rb], acc.at[dst_b.at[r]], sem_s.at[rb],
                         add=True)

    def _wait_scatter(k):
        r = lax.rem(k, 4)
        rb = lax.rem(k, 2)
        pltpu.make_async_copy(rows.at[rb], acc.at[dst_b.at[r]],
                              sem_s.at[rb]).wait()

    for s in range(2):
        base = (s * 2 + c) * NP

        pltpu.sync_copy(zeros_hbm, rows.at[0])

        def _zacc(k, _):
            pltpu.sync_copy(rows.at[0], acc.at[pl.ds(t * ROWS_PT + k * CH,
                                                     CH)])
            return 0

        lax.fori_loop(0, ROWS_PT // CH, _zacc, 0)
        plsc.subcore_barrier()

        _issue_idx(s, 0)
        _issue_idx(s, 1)
        _issue_idx(s, 2)
        _wait_idx(0)
        _issue_gather(0)

        def _edge(k, _):
            @pl.when(k >= 1)
            def _():
                _wait_scatter(k - 1)

            @pl.when(k + 3 < NCH)
            def _():
                _issue_idx(s, k + 3)

            @pl.when(k + 1 < NCH)
            def _():
                _wait_idx(k + 1)
                _issue_gather(k + 1)

            _wait_gather(k)
            _issue_scatter(k)
            return 0

        lax.fori_loop(0, NCH, _edge, 0)
        _wait_scatter(NCH - 1)
        plsc.subcore_barrier()

        def _flush(k, _):
            pltpu.sync_copy(acc.at[pl.ds(t * ROWS_PT + k * CH, CH)],
                            rows.at[0])
            pltpu.sync_copy(
                rows.at[0],
                out_hbm.at[pl.ds(base + t * ROWS_PT + k * CH, CH)])
            return 0

        lax.fori_loop(0, ROWS_PT // CH, _flush, 0)
        plsc.subcore_barrier()


def _scale_body(deg0, deg1, s1, s2, y, dinv):
    d = deg0[...] + deg1[...] + 1.0
    di = lax.rsqrt(d)
    y1 = di * s1[...]
    y2 = di * s2[...]
    y[0] = y1[:, :128]
    y[1] = y1[:, 128:]
    y[2] = y2[:, :128]
    y[3] = y2[:, 128:]
    dinv[...] = di


def _final_body(agg, s1, s2, h3, dinv, maskf, wg, bg, wl, bl, sc1, sc2):
    di = dinv[...]
    a1 = jnp.concatenate([agg[0], agg[1]], axis=1)
    a2 = jnp.concatenate([agg[2], agg[3]], axis=1)
    di2 = di * di
    z1 = di * a1 + di2 * s1[...]
    z2 = di * a2 + di2 * s2[...]
    wgm = wg[...]
    bgv = bg[...]
    g1 = jnp.maximum(jnp.dot(z1, wgm, preferred_element_type=jnp.float32)
                     + bgv, 0.0)
    g2 = jnp.maximum(jnp.dot(z2, wgm, preferred_element_type=jnp.float32)
                     + bgv, 0.0)
    wv = jnp.sum(wl[...], axis=1, keepdims=True)
    bs = jnp.sum(bl[...])
    t1 = jnp.dot(g1, wv, preferred_element_type=jnp.float32) + bs
    t2 = jnp.dot(g2, wv, preferred_element_type=jnp.float32) + bs
    t3 = jnp.dot(h3[...], wv, preferred_element_type=jnp.float32) + bs
    sc1[...] = t1
    sc2[...] = jnp.where(maskf[...] > 0.5, t3, t2)


def kernel(seq1, seq2, h_3, edge_index, W_gcn, b_gcn, W_lin, b_lin):
    src = edge_index[0]
    dst = edge_index[1]
    srcp = jnp.concatenate([src, jnp.zeros((EP - E,), jnp.int32)])
    dstf = jnp.concatenate([dst, jnp.full((EP - E,), N, jnp.int32)])
    dstr = dstf.reshape(EP // CH, CH)
    soff = (jnp.arange(4, dtype=jnp.int32)[:, None] * N
            + srcp[None, :]).reshape(-1)

    mesh = plsc.VectorSubcoreMesh(core_axis_name="c", subcore_axis_name="s")

    deg_call = functools.partial(
        pl.kernel,
        mesh=mesh,
        out_type=jax.ShapeDtypeStruct((2 * NP,), jnp.float32),
        scratch_types=[
            pltpu.VMEM_SHARED((NP,), jnp.float32),
            pltpu.VMEM((40, CH), jnp.int32),
            pltpu.VMEM((ROWS_PT,), jnp.float32),
            pltpu.VMEM((CH,), jnp.float32),
            pltpu.SemaphoreType.DMA,
        ],
    )(_deg_body)
    deg_flat = deg_call(dstr)
    deg0 = deg_flat[:N].reshape(N, 1)
    deg1 = deg_flat[NP:NP + N].reshape(N, 1)

    BR = 400
    nblk = N // BR
    y4, dinv = pl.pallas_call(
        _scale_body,
        grid=(nblk,),
        in_specs=[
            pl.BlockSpec((BR, 1), lambda i: (i, 0)),
            pl.BlockSpec((BR, 1), lambda i: (i, 0)),
            pl.BlockSpec((BR, 256), lambda i: (i, 0)),
            pl.BlockSpec((BR, 256), lambda i: (i, 0)),
        ],
        out_specs=[
            pl.BlockSpec((4, BR, 128), lambda i: (0, i, 0)),
            pl.BlockSpec((BR, 1), lambda i: (i, 0)),
        ],
        out_shape=[
            jax.ShapeDtypeStruct((4, N, 128), jnp.float32),
            jax.ShapeDtypeStruct((N, 1), jnp.float32),
        ],
    )(deg0, deg1, seq1, seq2)

    yflat = y4.reshape(4 * N, 128)

    agg_call = functools.partial(
        pl.kernel,
        mesh=mesh,
        out_type=jax.ShapeDtypeStruct((4 * NP, 128), jnp.float32),
        scratch_types=[
            pltpu.VMEM_SHARED((NP, 128), jnp.float32),
            pltpu.VMEM((4, CH), jnp.int32),
            pltpu.VMEM((4, CH), jnp.int32),
            pltpu.VMEM((2, CH, 128), jnp.float32),
            pltpu.SemaphoreType.DMA((2,)),
            pltpu.SemaphoreType.DMA((2,)),
            pltpu.SemaphoreType.DMA((4,)),
            pltpu.SemaphoreType.DMA((4,)),
        ],
    )(_agg_body)
    zeros128 = jnp.zeros((CH, 128), jnp.float32)
    agg = agg_call(yflat, soff, dstf, zeros128).reshape(4, NP, 128)

    maskf = (jax.random.uniform(jax.random.key(42), (N,), dtype=jnp.float32)
             > 0.5).astype(jnp.float32).reshape(N, 1)

    sc1, sc2 = pl.pallas_call(
        _final_body,
        grid=(nblk,),
        in_specs=[
            pl.BlockSpec((4, BR, 128), lambda i: (0, i, 0)),
            pl.BlockSpec((BR, 256), lambda i: (i, 0)),
            pl.BlockSpec((BR, 256), lambda i: (i, 0)),
            pl.BlockSpec((BR, 256), lambda i: (i, 0)),
            pl.BlockSpec((BR, 1), lambda i: (i, 0)),
            pl.BlockSpec((BR, 1), lambda i: (i, 0)),
            pl.BlockSpec((256, 256), lambda i: (0, 0)),
            pl.BlockSpec((1, 256), lambda i: (0, 0)),
            pl.BlockSpec((256, 256), lambda i: (0, 0)),
            pl.BlockSpec((1, 256), lambda i: (0, 0)),
        ],
        out_specs=[
            pl.BlockSpec((BR, 1), lambda i: (i, 0)),
            pl.BlockSpec((BR, 1), lambda i: (i, 0)),
        ],
        out_shape=[
            jax.ShapeDtypeStruct((N, 1), jnp.float32),
            jax.ShapeDtypeStruct((N, 1), jnp.float32),
        ],
    )(agg, seq1, seq2, h_3, dinv, maskf, W_gcn,
      b_gcn.reshape(1, D), W_lin, b_lin.reshape(1, D))

    return jnp.concatenate([sc1[:, 0], sc2[:, 0]])


# speedup vs baseline: 1.0006x; 1.0006x over previous
import functools

import jax
import jax.numpy as jnp
from jax import lax
from jax.experimental import pallas as pl
from jax.experimental.pallas import tpu as pltpu
from jax.experimental.pallas import tpu_sc as plsc

N = 10000
D = 256
E = 160000
NP = 10240
EP = 163840
CH = 128
NT = 16
ROWS_PT = NP // NT


def _deg_body(dstr, out_hbm, acc, dst_a, tmp, ones_v, sem):
    del sem
    c = lax.axis_index("c")
    t = lax.axis_index("s")
    w = c * NT + t

    def _fill_zero(i, _):
        tmp[pl.ds(i * 16, 16)] = jnp.zeros((16,), jnp.float32)
        return 0

    lax.fori_loop(0, ROWS_PT // 16, _fill_zero, 0)

    def _fill_one(i, _):
        ones_v[pl.ds(i * 16, 16)] = jnp.ones((16,), jnp.float32)
        return 0

    lax.fori_loop(0, CH // 16, _fill_one, 0)

    pltpu.sync_copy(tmp, acc.at[pl.ds(t * ROWS_PT, ROWS_PT)])
    pltpu.sync_copy(dstr.at[pl.ds(w * 40, 40)], dst_a)
    plsc.subcore_barrier()

    def _scat(j, _):
        pltpu.sync_copy(ones_v, acc.at[dst_a.at[j]], add=True)
        return 0

    lax.fori_loop(0, 40, _scat, 0)
    plsc.subcore_barrier()

    pltpu.sync_copy(acc.at[pl.ds(t * ROWS_PT, ROWS_PT)], tmp)
    pltpu.sync_copy(tmp, out_hbm.at[pl.ds(c * NP + t * ROWS_PT, ROWS_PT)])


def _agg_body(y_hbm, soff_hbm, dstf, zeros_hbm, out_hbm, acc, src_b, dst_b,
              rows, sem_g, sem_s, sem_si, sem_di):
    c = lax.axis_index("c")
    t = lax.axis_index("s")
    NCH = 80

    def _issue_idx(s, k):
        r = lax.rem(k, 4)
        sbase = (s * 2 + c) * EP
        eoff = (t * NCH + k) * CH
        pltpu.async_copy(soff_hbm.at[pl.ds(sbase + eoff, CH)],
                         src_b.at[r], sem_si.at[r])
        pltpu.async_copy(dstf.at[pl.ds(eoff, CH)], dst_b.at[r], sem_di.at[r])

    def _wait_idx(k):
        r = lax.rem(k, 4)
        pltpu.make_async_copy(soff_hbm.at[pl.ds(0, CH)], src_b.at[r],
                              sem_si.at[r]).wait()
        pltpu.make_async_copy(dstf.at[pl.ds(0, CH)], dst_b.at[r],
                              sem_di.at[r]).wait()

    def _issue_gather(k):
        r = lax.rem(k, 4)
        rb = lax.rem(k, 2)
        pltpu.async_copy(y_hbm.at[src_b.at[r]], rows.at[rb], sem_g.at[rb])

    def _wait_gather(k):
        r = lax.rem(k, 4)
        rb = lax.rem(k, 2)
        pltpu.make_async_copy(y_hbm.at[src_b.at[r]], rows.at[rb],
                              sem_g.at[rb]).wait()

    def _issue_scatter(k):
        r = lax.rem(k, 4)
        rb = lax.rem(k, 2)
        pltpu.async_copy(rows.at[rb], acc.at[dst_b.at[r]], sem_s.at[rb],
                         add=True)

    def _wait_scatter(k):
        r = lax.rem(k, 4)
        rb = lax.rem(k, 2)
        pltpu.make_async_copy(rows.at[rb], acc.at[dst_b.at[r]],
                              sem_s.at[rb]).wait()

    for s in range(2):
        base = (s * 2 + c) * NP

        pltpu.sync_copy(zeros_hbm, rows.at[0])

        def _zacc(k, _):
            pltpu.sync_copy(rows.at[0], acc.at[pl.ds(t * ROWS_PT + k * CH,
                                                     CH)])
            return 0

        lax.fori_loop(0, ROWS_PT // CH, _zacc, 0)
        plsc.subcore_barrier()

        _issue_idx(s, 0)
        _issue_idx(s, 1)
        _issue_idx(s, 2)
        _wait_idx(0)
        _issue_gather(0)

        def _edge(k, _):
            @pl.when(k + 3 < NCH)
            def _():
                _issue_idx(s, k + 3)

            @pl.when(k + 1 < NCH)
            def _():
                _wait_idx(k + 1)
                _issue_gather(k + 1)

            _wait_gather(k)
            rb = lax.rem(k, 2)
            r = lax.rem(k, 4)
            pltpu.sync_copy(rows.at[rb], acc.at[dst_b.at[r]], add=True)
            return 0

        lax.fori_loop(0, NCH, _edge, 0)
        plsc.subcore_barrier()

        def _flush(k, _):
            pltpu.sync_copy(acc.at[pl.ds(t * ROWS_PT + k * CH, CH)],
                            rows.at[0])
            pltpu.sync_copy(
                rows.at[0],
                out_hbm.at[pl.ds(base + t * ROWS_PT + k * CH, CH)])
            return 0

        lax.fori_loop(0, ROWS_PT // CH, _flush, 0)
        plsc.subcore_barrier()


def _scale_body(deg0, deg1, s1, s2, y, dinv):
    d = deg0[...] + deg1[...] + 1.0
    di = lax.rsqrt(d)
    y1 = di * s1[...]
    y2 = di * s2[...]
    y[0] = y1[:, :128]
    y[1] = y1[:, 128:]
    y[2] = y2[:, :128]
    y[3] = y2[:, 128:]
    dinv[...] = di


def _final_body(agg, s1, s2, h3, dinv, maskf, wg, bg, wl, bl, sc1, sc2):
    di = dinv[...]
    a1 = jnp.concatenate([agg[0], agg[1]], axis=1)
    a2 = jnp.concatenate([agg[2], agg[3]], axis=1)
    di2 = di * di
    z1 = di * a1 + di2 * s1[...]
    z2 = di * a2 + di2 * s2[...]
    wgm = wg[...]
    bgv = bg[...]
    g1 = jnp.maximum(jnp.dot(z1, wgm, preferred_element_type=jnp.float32)
                     + bgv, 0.0)
    g2 = jnp.maximum(jnp.dot(z2, wgm, preferred_element_type=jnp.float32)
                     + bgv, 0.0)
    wv = jnp.sum(wl[...], axis=1, keepdims=True)
    bs = jnp.sum(bl[...])
    t1 = jnp.dot(g1, wv, preferred_element_type=jnp.float32) + bs
    t2 = jnp.dot(g2, wv, preferred_element_type=jnp.float32) + bs
    t3 = jnp.dot(h3[...], wv, preferred_element_type=jnp.float32) + bs
    sc1[...] = t1
    sc2[...] = jnp.where(maskf[...] > 0.5, t3, t2)


def kernel(seq1, seq2, h_3, edge_index, W_gcn, b_gcn, W_lin, b_lin):
    src = edge_index[0]
    dst = edge_index[1]
    srcp = jnp.concatenate([src, jnp.zeros((EP - E,), jnp.int32)])
    dstf = jnp.concatenate([dst, jnp.full((EP - E,), N, jnp.int32)])
    dstr = dstf.reshape(EP // CH, CH)
    soff = (jnp.arange(4, dtype=jnp.int32)[:, None] * N
            + srcp[None, :]).reshape(-1)

    mesh = plsc.VectorSubcoreMesh(core_axis_name="c", subcore_axis_name="s")

    deg_call = functools.partial(
        pl.kernel,
        mesh=mesh,
        out_type=jax.ShapeDtypeStruct((2 * NP,), jnp.float32),
        scratch_types=[
            pltpu.VMEM_SHARED((NP,), jnp.float32),
            pltpu.VMEM((40, CH), jnp.int32),
            pltpu.VMEM((ROWS_PT,), jnp.float32),
            pltpu.VMEM((CH,), jnp.float32),
            pltpu.SemaphoreType.DMA,
        ],
    )(_deg_body)
    deg_flat = deg_call(dstr)
    deg0 = deg_flat[:N].reshape(N, 1)
    deg1 = deg_flat[NP:NP + N].reshape(N, 1)

    BR = 400
    nblk = N // BR
    y4, dinv = pl.pallas_call(
        _scale_body,
        grid=(nblk,),
        in_specs=[
            pl.BlockSpec((BR, 1), lambda i: (i, 0)),
            pl.BlockSpec((BR, 1), lambda i: (i, 0)),
            pl.BlockSpec((BR, 256), lambda i: (i, 0)),
            pl.BlockSpec((BR, 256), lambda i: (i, 0)),
        ],
        out_specs=[
            pl.BlockSpec((4, BR, 128), lambda i: (0, i, 0)),
            pl.BlockSpec((BR, 1), lambda i: (i, 0)),
        ],
        out_shape=[
            jax.ShapeDtypeStruct((4, N, 128), jnp.float32),
            jax.ShapeDtypeStruct((N, 1), jnp.float32),
        ],
    )(deg0, deg1, seq1, seq2)

    yflat = y4.reshape(4 * N, 128)

    agg_call = functools.partial(
        pl.kernel,
        mesh=mesh,
        out_type=jax.ShapeDtypeStruct((4 * NP, 128), jnp.float32),
        scratch_types=[
            pltpu.VMEM_SHARED((NP, 128), jnp.float32),
            pltpu.VMEM((4, CH), jnp.int32),
            pltpu.VMEM((4, CH), jnp.int32),
            pltpu.VMEM((2, CH, 128), jnp.float32),
            pltpu.SemaphoreType.DMA((2,)),
            pltpu.SemaphoreType.DMA((2,)),
            pltpu.SemaphoreType.DMA((4,)),
            pltpu.SemaphoreType.DMA((4,)),
        ],
    )(_agg_body)
    zeros128 = jnp.zeros((CH, 128), jnp.float32)
    agg = agg_call(yflat, soff, dstf, zeros128).reshape(4, NP, 128)

    maskf = (jax.random.uniform(jax.random.key(42), (N,), dtype=jnp.float32)
             > 0.5).astype(jnp.float32).reshape(N, 1)

    sc1, sc2 = pl.pallas_call(
        _final_body,
        grid=(nblk,),
        in_specs=[
            pl.BlockSpec((4, BR, 128), lambda i: (0, i, 0)),
            pl.BlockSpec((BR, 256), lambda i: (i, 0)),
            pl.BlockSpec((BR, 256), lambda i: (i, 0)),
            pl.BlockSpec((BR, 256), lambda i: (i, 0)),
            pl.BlockSpec((BR, 1), lambda i: (i, 0)),
            pl.BlockSpec((BR, 1), lambda i: (i, 0)),
            pl.BlockSpec((256, 256), lambda i: (0, 0)),
            pl.BlockSpec((1, 256), lambda i: (0, 0)),
            pl.BlockSpec((256, 256), lambda i: (0, 0)),
            pl.BlockSpec((1, 256), lambda i: (0, 0)),
        ],
        out_specs=[
            pl.BlockSpec((BR, 1), lambda i: (i, 0)),
            pl.BlockSpec((BR, 1), lambda i: (i, 0)),
        ],
        out_shape=[
            jax.ShapeDtypeStruct((N, 1), jnp.float32),
            jax.ShapeDtypeStruct((N, 1), jnp.float32),
        ],
    )(agg, seq1, seq2, h_3, dinv, maskf, W_gcn,
      b_gcn.reshape(1, D), W_lin, b_lin.reshape(1, D))

    return jnp.concatenate([sc1[:, 0], sc2[:, 0]])


# speedup vs baseline: 1.0254x; 1.0249x over previous
import functools

import jax
import jax.numpy as jnp
from jax import lax
from jax.experimental import pallas as pl
from jax.experimental.pallas import tpu as pltpu
from jax.experimental.pallas import tpu_sc as plsc

N = 10000
D = 256
E = 160000
NP = 10240
EP = 163840
CH = 128
NT = 16
ROWS_PT = NP // NT


def _deg_body(dstr, out_hbm, acc, dst_a, tmp, ones_v, sem):
    del sem
    c = lax.axis_index("c")
    t = lax.axis_index("s")
    w = c * NT + t

    def _fill_zero(i, _):
        tmp[pl.ds(i * 16, 16)] = jnp.zeros((16,), jnp.float32)
        return 0

    lax.fori_loop(0, ROWS_PT // 16, _fill_zero, 0)

    def _fill_one(i, _):
        ones_v[pl.ds(i * 16, 16)] = jnp.ones((16,), jnp.float32)
        return 0

    lax.fori_loop(0, CH // 16, _fill_one, 0)

    pltpu.sync_copy(tmp, acc.at[pl.ds(t * ROWS_PT, ROWS_PT)])
    pltpu.sync_copy(dstr.at[pl.ds(w * 40, 40)], dst_a)
    plsc.subcore_barrier()

    def _scat(j, _):
        pltpu.sync_copy(ones_v, acc.at[dst_a.at[j]], add=True)
        return 0

    lax.fori_loop(0, 40, _scat, 0)
    plsc.subcore_barrier()

    pltpu.sync_copy(acc.at[pl.ds(t * ROWS_PT, ROWS_PT)], tmp)
    pltpu.sync_copy(tmp, out_hbm.at[pl.ds(c * NP + t * ROWS_PT, ROWS_PT)])


def _agg_body(y_hbm, soff_hbm, dstr, out_hbm, acc, src_b, dst_a, rows,
              sem_g, sem_si):
    c = lax.axis_index("c")
    t = lax.axis_index("s")
    NCH = 80

    def _issue_idx(s, k):
        r = lax.rem(k, 4)
        sbase = (s * 2 + c) * EP
        pltpu.async_copy(soff_hbm.at[pl.ds(sbase + (t * NCH + k) * CH, CH)],
                         src_b.at[r], sem_si.at[r])

    def _wait_idx(k):
        r = lax.rem(k, 4)
        pltpu.make_async_copy(soff_hbm.at[pl.ds(0, CH)], src_b.at[r],
                              sem_si.at[r]).wait()

    def _issue_gather(k):
        r = lax.rem(k, 4)
        rb = lax.rem(k, 2)
        pltpu.async_copy(y_hbm.at[src_b.at[r]], rows.at[rb], sem_g.at[rb])

    def _wait_gather(k):
        r = lax.rem(k, 4)
        rb = lax.rem(k, 2)
        pltpu.make_async_copy(y_hbm.at[src_b.at[r]], rows.at[rb],
                              sem_g.at[rb]).wait()

    pltpu.sync_copy(dstr.at[pl.ds(t * NCH, NCH)], dst_a)

    for s in range(2):
        pltpu.sync_copy(y_hbm.at[pl.ds(N, CH)], rows.at[0])

        def _zacc(k, _):
            pltpu.sync_copy(rows.at[0], acc.at[pl.ds(t * ROWS_PT + k * CH,
                                                     CH)])
            return 0

        lax.fori_loop(0, ROWS_PT // CH, _zacc, 0)
        plsc.subcore_barrier()

        _issue_idx(s, 0)
        _issue_idx(s, 1)
        _issue_idx(s, 2)
        _wait_idx(0)
        _issue_gather(0)

        def _edge(k, _):
            @pl.when(k + 3 < NCH)
            def _():
                _issue_idx(s, k + 3)

            @pl.when(k + 1 < NCH)
            def _():
                _wait_idx(k + 1)
                _issue_gather(k + 1)

            _wait_gather(k)
            rb = lax.rem(k, 2)
            pltpu.sync_copy(rows.at[rb], acc.at[dst_a.at[k]], add=True)
            return 0

        lax.fori_loop(0, NCH, _edge, 0)
        plsc.subcore_barrier()

        def _flush(k, _):
            pltpu.sync_copy(acc.at[pl.ds(t * ROWS_PT + k * CH, CH)],
                            rows.at[0])
            pltpu.sync_copy(
                rows.at[0],
                out_hbm.at[pl.ds((s * 2 + c) * NP + t * ROWS_PT + k * CH,
                                 CH)])
            return 0

        lax.fori_loop(0, ROWS_PT // CH, _flush, 0)
        plsc.subcore_barrier()


def _scale_body(deg0, deg1, s1, s2, y, dinv):
    i = pl.program_id(0)
    d = deg0[...] + deg1[...] + 1.0
    di = lax.rsqrt(d)
    ridx = lax.broadcasted_iota(jnp.int32, (256, 1), 0) + i * 256
    div = jnp.where(ridx < N, di, 0.0)
    y1 = div * s1[...]
    y2 = div * s2[...]
    y[0] = y1[:, :128]
    y[1] = y1[:, 128:]
    y[2] = y2[:, :128]
    y[3] = y2[:, 128:]
    dinv[...] = div


def _final_body(agg, s1, s2, h3, dinv, maskf, wg, bg, wl, bl, sc1, sc2):
    di = dinv[...]
    a1 = jnp.concatenate([agg[0], agg[1]], axis=1)
    a2 = jnp.concatenate([agg[2], agg[3]], axis=1)
    di2 = di * di
    z1 = di * a1 + di2 * s1[...]
    z2 = di * a2 + di2 * s2[...]
    wgm = wg[...]
    bgv = bg[...]
    g1 = jnp.maximum(jnp.dot(z1, wgm, preferred_element_type=jnp.float32)
                     + bgv, 0.0)
    g2 = jnp.maximum(jnp.dot(z2, wgm, preferred_element_type=jnp.float32)
                     + bgv, 0.0)
    wv = jnp.sum(wl[...], axis=1, keepdims=True)
    bs = jnp.sum(bl[...])
    t1 = jnp.dot(g1, wv, preferred_element_type=jnp.float32) + bs
    t2 = jnp.dot(g2, wv, preferred_element_type=jnp.float32) + bs
    t3 = jnp.dot(h3[...], wv, preferred_element_type=jnp.float32) + bs
    sc1[...] = t1
    sc2[...] = jnp.where(maskf[...] > 0.5, t3, t2)


def kernel(seq1, seq2, h_3, edge_index, W_gcn, b_gcn, W_lin, b_lin):
    src = edge_index[0]
    dst = edge_index[1]
    pad = jnp.full((EP - E,), N, jnp.int32)
    srcp = jnp.concatenate([src, pad])
    dstf = jnp.concatenate([dst, pad])
    dstr2 = dstf.reshape(EP // CH, CH)
    soff = (jnp.arange(4, dtype=jnp.int32)[:, None] * NP
            + srcp[None, :]).reshape(-1)

    mesh = plsc.VectorSubcoreMesh(core_axis_name="c", subcore_axis_name="s")

    deg_call = functools.partial(
        pl.kernel,
        mesh=mesh,
        out_type=jax.ShapeDtypeStruct((2 * NP,), jnp.float32),
        scratch_types=[
            pltpu.VMEM_SHARED((NP,), jnp.float32),
            pltpu.VMEM((40, CH), jnp.int32),
            pltpu.VMEM((ROWS_PT,), jnp.float32),
            pltpu.VMEM((CH,), jnp.float32),
            pltpu.SemaphoreType.DMA,
        ],
    )(_deg_body)
    deg_flat = deg_call(dstr2)
    deg0 = deg_flat[:NP].reshape(NP, 1)
    deg1 = deg_flat[NP:].reshape(NP, 1)

    seq1p = jnp.pad(seq1, ((0, NP - N), (0, 0)))
    seq2p = jnp.pad(seq2, ((0, NP - N), (0, 0)))

    nblk = NP // 256
    y4, dinv = pl.pallas_call(
        _scale_body,
        grid=(nblk,),
        in_specs=[
            pl.BlockSpec((256, 1), lambda i: (i, 0)),
            pl.BlockSpec((256, 1), lambda i: (i, 0)),
            pl.BlockSpec((256, 256), lambda i: (i, 0)),
            pl.BlockSpec((256, 256), lambda i: (i, 0)),
        ],
        out_specs=[
            pl.BlockSpec((4, 256, 128), lambda i: (0, i, 0)),
            pl.BlockSpec((256, 1), lambda i: (i, 0)),
        ],
        out_shape=[
            jax.ShapeDtypeStruct((4, NP, 128), jnp.float32),
            jax.ShapeDtypeStruct((NP, 1), jnp.float32),
        ],
    )(deg0, deg1, seq1p, seq2p)

    yflat = y4.reshape(4 * NP, 128)

    agg_call = functools.partial(
        pl.kernel,
        mesh=mesh,
        out_type=jax.ShapeDtypeStruct((4 * NP, 128), jnp.float32),
        scratch_types=[
            pltpu.VMEM_SHARED((NP, 128), jnp.float32),
            pltpu.VMEM((4, CH), jnp.int32),
            pltpu.VMEM((80, CH), jnp.int32),
            pltpu.VMEM((2, CH, 128), jnp.float32),
            pltpu.SemaphoreType.DMA((2,)),
            pltpu.SemaphoreType.DMA((4,)),
        ],
    )(_agg_body)
    agg = agg_call(yflat, soff, dstr2).reshape(4, NP, 128)

    maskf = (jax.random.uniform(jax.random.key(42), (N,), dtype=jnp.float32)
             > 0.5).astype(jnp.float32)
    maskp = jnp.pad(maskf, (0, NP - N)).reshape(NP, 1)
    h3p = jnp.pad(h_3, ((0, NP - N), (0, 0)))

    sc1, sc2 = pl.pallas_call(
        _final_body,
        grid=(nblk,),
        in_specs=[
            pl.BlockSpec((4, 256, 128), lambda i: (0, i, 0)),
            pl.BlockSpec((256, 256), lambda i: (i, 0)),
            pl.BlockSpec((256, 256), lambda i: (i, 0)),
            pl.BlockSpec((256, 256), lambda i: (i, 0)),
            pl.BlockSpec((256, 1), lambda i: (i, 0)),
            pl.BlockSpec((256, 1), lambda i: (i, 0)),
            pl.BlockSpec((256, 256), lambda i: (0, 0)),
            pl.BlockSpec((1, 256), lambda i: (0, 0)),
            pl.BlockSpec((256, 256), lambda i: (0, 0)),
            pl.BlockSpec((1, 256), lambda i: (0, 0)),
        ],
        out_specs=[
            pl.BlockSpec((256, 1), lambda i: (i, 0)),
            pl.BlockSpec((256, 1), lambda i: (i, 0)),
        ],
        out_shape=[
            jax.ShapeDtypeStruct((NP, 1), jnp.float32),
            jax.ShapeDtypeStruct((NP, 1), jnp.float32),
        ],
    )(agg, seq1p, seq2p, h3p, dinv, maskp, W_gcn,
      b_gcn.reshape(1, D), W_lin, b_lin.reshape(1, D))

    return jnp.concatenate([sc1[:N, 0], sc2[:N, 0]])


# speedup vs baseline: 1.1177x; 1.0900x over previous
import functools

import jax
import jax.numpy as jnp
from jax import lax
from jax.experimental import pallas as pl
from jax.experimental.pallas import tpu as pltpu
from jax.experimental.pallas import tpu_sc as plsc

N = 10000
D = 256
E = 160000
NP = 10240
EP = 163840
CH = 128
NT = 16
ROWS_PT = NP // NT


def _deg_body(dstr, out_hbm, acc, dst_a, tmp, ones_v, sem):
    del sem
    c = lax.axis_index("c")
    t = lax.axis_index("s")
    w = c * NT + t

    def _fill_zero(i, _):
        tmp[pl.ds(i * 16, 16)] = jnp.zeros((16,), jnp.float32)
        return 0

    lax.fori_loop(0, ROWS_PT // 16, _fill_zero, 0)

    def _fill_one(i, _):
        ones_v[pl.ds(i * 16, 16)] = jnp.ones((16,), jnp.float32)
        return 0

    lax.fori_loop(0, CH // 16, _fill_one, 0)

    pltpu.sync_copy(tmp, acc.at[pl.ds(t * ROWS_PT, ROWS_PT)])
    pltpu.sync_copy(dstr.at[pl.ds(w * 40, 40)], dst_a)
    plsc.subcore_barrier()

    def _scat(j, _):
        pltpu.sync_copy(ones_v, acc.at[dst_a.at[j]], add=True)
        return 0

    lax.fori_loop(0, 40, _scat, 0)
    plsc.subcore_barrier()

    pltpu.sync_copy(acc.at[pl.ds(t * ROWS_PT, ROWS_PT)], tmp)
    pltpu.sync_copy(tmp, out_hbm.at[pl.ds(c * NP + t * ROWS_PT, ROWS_PT)])


def _agg_body(y_hbm, soff_hbm, dstf_hbm, out_hbm, acc, src_b, dst_b, rows,
              sem_g, sem_si, sem_di):
    c = lax.axis_index("c")
    t = lax.axis_index("s")
    NCH = 80

    def _issue_idx(s, k):
        r = lax.rem(k, 4)
        sbase = (s * 2 + c) * EP
        eoff = (t * NCH + k) * CH
        pltpu.async_copy(soff_hbm.at[pl.ds(sbase + eoff, CH)],
                         src_b.at[r], sem_si.at[r])
        pltpu.async_copy(dstf_hbm.at[pl.ds(eoff, CH)], dst_b.at[r],
                         sem_di.at[r])

    def _wait_idx(k):
        r = lax.rem(k, 4)
        pltpu.make_async_copy(soff_hbm.at[pl.ds(0, CH)], src_b.at[r],
                              sem_si.at[r]).wait()
        pltpu.make_async_copy(dstf_hbm.at[pl.ds(0, CH)], dst_b.at[r],
                              sem_di.at[r]).wait()

    def _issue_gather(k):
        r = lax.rem(k, 4)
        rb = lax.rem(k, 2)
        pltpu.async_copy(y_hbm.at[src_b.at[r]], rows.at[rb], sem_g.at[rb])

    def _wait_gather(k):
        r = lax.rem(k, 4)
        rb = lax.rem(k, 2)
        pltpu.make_async_copy(y_hbm.at[src_b.at[r]], rows.at[rb],
                              sem_g.at[rb]).wait()

    for s in range(2):
        pltpu.sync_copy(y_hbm.at[pl.ds(N, CH)], rows.at[0])

        def _zacc(k, _):
            pltpu.sync_copy(rows.at[0], acc.at[pl.ds(t * ROWS_PT + k * CH,
                                                     CH)])
            return 0

        lax.fori_loop(0, ROWS_PT // CH, _zacc, 0)
        plsc.subcore_barrier()

        _issue_idx(s, 0)
        _issue_idx(s, 1)
        _issue_idx(s, 2)
        _wait_idx(0)
        _issue_gather(0)

        def _edge(k, _):
            @pl.when(k + 3 < NCH)
            def _():
                _issue_idx(s, k + 3)

            @pl.when(k + 1 < NCH)
            def _():
                _wait_idx(k + 1)
                _issue_gather(k + 1)

            _wait_gather(k)
            rb = lax.rem(k, 2)
            r = lax.rem(k, 4)
            pltpu.sync_copy(rows.at[rb], acc.at[dst_b.at[r]], add=True)
            return 0

        lax.fori_loop(0, NCH, _edge, 0)
        plsc.subcore_barrier()

        def _flush(k, _):
            pltpu.sync_copy(acc.at[pl.ds(t * ROWS_PT + k * CH, CH)],
                            rows.at[0])
            pltpu.sync_copy(
                rows.at[0],
                out_hbm.at[pl.ds((s * 2 + c) * NP + t * ROWS_PT + k * CH,
                                 CH)])
            return 0

        lax.fori_loop(0, ROWS_PT // CH, _flush, 0)
        plsc.subcore_barrier()


def _scale_body(deg0, deg1, s1, s2, y, dinv):
    i = pl.program_id(0)
    d = deg0[...] + deg1[...] + 1.0
    di = lax.rsqrt(d)
    ridx = lax.broadcasted_iota(jnp.int32, (256, 1), 0) + i * 256
    div = jnp.where(ridx < N, di, 0.0)
    y1 = div * s1[...]
    y2 = div * s2[...]
    y[0] = y1[:, :128]
    y[1] = y1[:, 128:]
    y[2] = y2[:, :128]
    y[3] = y2[:, 128:]
    dinv[...] = div


def _final_body(agg, s1, s2, h3, dinv, maskf, wg, bg, wl, bl, sc1, sc2):
    di = dinv[...]
    a1 = jnp.concatenate([agg[0], agg[1]], axis=1)
    a2 = jnp.concatenate([agg[2], agg[3]], axis=1)
    di2 = di * di
    z1 = di * a1 + di2 * s1[...]
    z2 = di * a2 + di2 * s2[...]
    wgm = wg[...]
    bgv = bg[...]
    g1 = jnp.maximum(jnp.dot(z1, wgm, preferred_element_type=jnp.float32)
                     + bgv, 0.0)
    g2 = jnp.maximum(jnp.dot(z2, wgm, preferred_element_type=jnp.float32)
                     + bgv, 0.0)
    wv = jnp.sum(wl[...], axis=1, keepdims=True)
    bs = jnp.sum(bl[...])
    t1 = jnp.dot(g1, wv, preferred_element_type=jnp.float32) + bs
    t2 = jnp.dot(g2, wv, preferred_element_type=jnp.float32) + bs
    t3 = jnp.dot(h3[...], wv, preferred_element_type=jnp.float32) + bs
    sc1[...] = t1
    sc2[...] = jnp.where(maskf[...] > 0.5, t3, t2)


def kernel(seq1, seq2, h_3, edge_index, W_gcn, b_gcn, W_lin, b_lin):
    src = edge_index[0]
    dst = edge_index[1]
    pad = jnp.full((EP - E,), N, jnp.int32)
    srcp = jnp.concatenate([src, pad])
    dstf = jnp.concatenate([dst, pad])
    dstr2 = dstf.reshape(EP // CH, CH)
    soff = (jnp.arange(4, dtype=jnp.int32)[:, None] * NP
            + srcp[None, :]).reshape(-1)

    mesh = plsc.VectorSubcoreMesh(core_axis_name="c", subcore_axis_name="s")

    deg_call = functools.partial(
        pl.kernel,
        mesh=mesh,
        out_type=jax.ShapeDtypeStruct((2 * NP,), jnp.float32),
        scratch_types=[
            pltpu.VMEM_SHARED((NP,), jnp.float32),
            pltpu.VMEM((40, CH), jnp.int32),
            pltpu.VMEM((ROWS_PT,), jnp.float32),
            pltpu.VMEM((CH,), jnp.float32),
            pltpu.SemaphoreType.DMA,
        ],
    )(_deg_body)
    deg_flat = deg_call(dstr2)
    deg0 = deg_flat[:NP].reshape(NP, 1)
    deg1 = deg_flat[NP:].reshape(NP, 1)

    seq1p = jnp.pad(seq1, ((0, NP - N), (0, 0)))
    seq2p = jnp.pad(seq2, ((0, NP - N), (0, 0)))

    nblk = NP // 256
    y4, dinv = pl.pallas_call(
        _scale_body,
        grid=(nblk,),
        in_specs=[
            pl.BlockSpec((256, 1), lambda i: (i, 0)),
            pl.BlockSpec((256, 1), lambda i: (i, 0)),
            pl.BlockSpec((256, 256), lambda i: (i, 0)),
            pl.BlockSpec((256, 256), lambda i: (i, 0)),
        ],
        out_specs=[
            pl.BlockSpec((4, 256, 128), lambda i: (0, i, 0)),
            pl.BlockSpec((256, 1), lambda i: (i, 0)),
        ],
        out_shape=[
            jax.ShapeDtypeStruct((4, NP, 128), jnp.float32),
            jax.ShapeDtypeStruct((NP, 1), jnp.float32),
        ],
    )(deg0, deg1, seq1p, seq2p)

    yflat = y4.reshape(4 * NP, 128)

    agg_call = functools.partial(
        pl.kernel,
        mesh=mesh,
        out_type=jax.ShapeDtypeStruct((4 * NP, 128), jnp.float32),
        scratch_types=[
            pltpu.VMEM_SHARED((NP, 128), jnp.float32),
            pltpu.VMEM((4, CH), jnp.int32),
            pltpu.VMEM((4, CH), jnp.int32),
            pltpu.VMEM((2, CH, 128), jnp.float32),
            pltpu.SemaphoreType.DMA((2,)),
            pltpu.SemaphoreType.DMA((4,)),
            pltpu.SemaphoreType.DMA((4,)),
        ],
    )(_agg_body)
    agg = agg_call(yflat, soff, dstf).reshape(4, NP, 128)

    maskf = (jax.random.uniform(jax.random.key(42), (N,), dtype=jnp.float32)
             > 0.5).astype(jnp.float32)
    maskp = jnp.pad(maskf, (0, NP - N)).reshape(NP, 1)
    h3p = jnp.pad(h_3, ((0, NP - N), (0, 0)))

    sc1, sc2 = pl.pallas_call(
        _final_body,
        grid=(nblk,),
        in_specs=[
            pl.BlockSpec((4, 256, 128), lambda i: (0, i, 0)),
            pl.BlockSpec((256, 256), lambda i: (i, 0)),
            pl.BlockSpec((256, 256), lambda i: (i, 0)),
            pl.BlockSpec((256, 256), lambda i: (i, 0)),
            pl.BlockSpec((256, 1), lambda i: (i, 0)),
            pl.BlockSpec((256, 1), lambda i: (i, 0)),
            pl.BlockSpec((256, 256), lambda i: (0, 0)),
            pl.BlockSpec((1, 256), lambda i: (0, 0)),
            pl.BlockSpec((256, 256), lambda i: (0, 0)),
            pl.BlockSpec((1, 256), lambda i: (0, 0)),
        ],
        out_specs=[
            pl.BlockSpec((256, 1), lambda i: (i, 0)),
            pl.BlockSpec((256, 1), lambda i: (i, 0)),
        ],
        out_shape=[
            jax.ShapeDtypeStruct((NP, 1), jnp.float32),
            jax.ShapeDtypeStruct((NP, 1), jnp.float32),
        ],
    )(agg, seq1p, seq2p, h3p, dinv, maskp, W_gcn,
      b_gcn.reshape(1, D), W_lin, b_lin.reshape(1, D))

    return jnp.concatenate([sc1[:N, 0], sc2[:N, 0]])
